# baseline (speedup 1.0000x reference)
import functools

import jax
import jax.numpy as jnp
from jax import lax
from jax.experimental import pallas as pl
from jax.experimental.pallas import tpu as pltpu
from jax.experimental.pallas import tpu_sc as plsc

_N = 10000
_NP = 10112
_H = 128
_G = 64
_NC = 2
_NS = 16
_K = 384
_NCHK = 28
_EPW = _K * _NCHK
_EPAD = _EPW * _NC * _NS

_mesh = plsc.VectorSubcoreMesh(core_axis_name="c", subcore_axis_name="s")


def _fill_rows(ref, nrows, width, value):
    @pl.loop(0, nrows)
    def _(r):
        for c in range(0, width, 16):
            ref[r, pl.ds(c, 16)] = jnp.full((16,), value, jnp.float32)


@functools.partial(
    pl.kernel,
    out_type=jax.ShapeDtypeStruct((_NC, _NP, _H), jnp.float32),
    mesh=_mesh,
    scratch_types=[
        pltpu.VMEM((_K,), jnp.int32),
        pltpu.VMEM((_K, _H), jnp.float32),
        pltpu.VMEM_SHARED((_NP, _H), jnp.float32),
    ],
)
def _deg_kernel(dst_hbm, z_hbm, out_hbm, idx_v, ones_v, sp):
    cid = lax.axis_index("c")
    sid = lax.axis_index("s")
    _fill_rows(ones_v, _K, _H, 1.0)

    @pl.when(sid == 0)
    def _():
        pltpu.sync_copy(z_hbm, sp)

    plsc.subcore_barrier()

    @pl.loop(0, _NCHK)
    def _(j):
        pltpu.sync_copy(dst_hbm.at[cid, sid, j], idx_v)
        pltpu.sync_copy(ones_v, sp.at[idx_v], add=True)

    plsc.subcore_barrier()

    @pl.when(sid == 1)
    def _():
        pltpu.sync_copy(sp, out_hbm.at[cid])


@functools.partial(
    pl.kernel,
    out_type=jax.ShapeDtypeStruct((_NC, _NP, _H), jnp.float32),
    mesh=_mesh,
    scratch_types=[
        pltpu.VMEM((_K,), jnp.int32),
        pltpu.VMEM((_K,), jnp.int32),
        pltpu.VMEM((_K, _H), jnp.float32),
        pltpu.VMEM_SHARED((_NP, _H), jnp.float32),
        pltpu.SemaphoreType.DMA,
    ],
)
def _agg_kernel(hs_hbm, src_hbm, dst_hbm, z_hbm, out_hbm,
                idxs_v, idxd_v, rows_v, sp, sem):
    cid = lax.axis_index("c")
    sid = lax.axis_index("s")

    @pl.when(sid == 0)
    def _():
        pltpu.sync_copy(z_hbm, sp)

    plsc.subcore_barrier()

    @pl.loop(0, _NCHK)
    def _(j):
        pltpu.sync_copy(src_hbm.at[cid, sid, j], idxs_v)
        pltpu.sync_copy(dst_hbm.at[cid, sid, j], idxd_v)
        pltpu.async_copy(hs_hbm.at[idxs_v], rows_v, sem).wait()
        pltpu.sync_copy(rows_v, sp.at[idxd_v], add=True)

    plsc.subcore_barrier()

    @pl.when(sid == 1)
    def _():
        pltpu.sync_copy(sp, out_hbm.at[cid])


def _first_body(x_ref, w_ref, deg_ref, hs_ref, dinv_ref):
    deg = deg_ref[0, :, 0:1] + deg_ref[1, :, 0:1]
    rows = lax.broadcasted_iota(jnp.int32, (_NP, 1), 0)
    dinv = jnp.where(rows < _N, lax.rsqrt(jnp.maximum(deg, 1e-12)), 0.0)
    m = jnp.dot(x_ref[...], w_ref[...],
                preferred_element_type=jnp.float32,
                precision=lax.Precision.HIGHEST)
    hs_ref[...] = dinv * m
    dinv_ref[...] = dinv


_first_tc = pl.pallas_call(
    _first_body,
    out_shape=(jax.ShapeDtypeStruct((_NP, _H), jnp.float32),
               jax.ShapeDtypeStruct((_NP, 1), jnp.float32)),
)


def _mid_body(agg_ref, dinv_ref, b_ref, w_ref, hs_ref):
    agg = agg_ref[0] + agg_ref[1]
    t = jnp.maximum(dinv_ref[...] * agg + b_ref[...], 0.0)
    m = jnp.dot(t, w_ref[...], preferred_element_type=jnp.float32,
                precision=lax.Precision.HIGHEST)
    hs_ref[...] = dinv_ref[...] * m


_mid_tc = pl.pallas_call(
    _mid_body,
    out_shape=jax.ShapeDtypeStruct((_NP, _H), jnp.float32),
)


def _final_body(agg_ref, dinv_ref, b_ref, batch_ref, wl_ref, bl_ref, out_ref):
    h3 = dinv_ref[...] * (agg_ref[0] + agg_ref[1]) + b_ref[...]
    gids = lax.broadcasted_iota(jnp.int32, (_G, _NP), 0)
    oh = (gids == batch_ref[...]).astype(jnp.float32)
    sums = jnp.dot(oh, h3, preferred_element_type=jnp.float32,
                   precision=lax.Precision.HIGHEST)
    cnt = jnp.sum(oh, axis=1, keepdims=True)
    pooled = sums / jnp.maximum(cnt, 1.0)
    out_ref[...] = (jnp.dot(pooled, wl_ref[...],
                            preferred_element_type=jnp.float32,
                            precision=lax.Precision.HIGHEST)
                    + bl_ref[...])


_final_tc = pl.pallas_call(
    _final_body,
    out_shape=jax.ShapeDtypeStruct((_G, 16), jnp.float32),
)


def kernel(x, edge_index, batch, W1, b1, W2, b2, W3, b3, Wl, bl):
    loop = jnp.arange(_N, dtype=jnp.int32)
    src = jnp.concatenate([edge_index[0].astype(jnp.int32), loop])
    dst = jnp.concatenate([edge_index[1].astype(jnp.int32), loop])
    pad = jnp.full((_EPAD - src.shape[0],), _N, dtype=jnp.int32)
    srcr = jnp.concatenate([src, pad]).reshape(_NC, _NS, _NCHK, _K)
    dstr = jnp.concatenate([dst, pad]).reshape(_NC, _NS, _NCHK, _K)
    x_pad = jnp.pad(x, ((0, _NP - _N), (0, 0)))
    batch_row = jnp.pad(batch.astype(jnp.int32), (0, _NP - _N),
                        constant_values=_G).reshape(1, _NP)
    zeros = jnp.zeros((_NP, _H), jnp.float32)

    degp = _deg_kernel(dstr, zeros)
    hs1, dinv = _first_tc(x_pad, W1, degp)
    agg1 = _agg_kernel(hs1, srcr, dstr, zeros)
    hs2 = _mid_tc(agg1, dinv, b1.reshape(1, _H), W2)
    agg2 = _agg_kernel(hs2, srcr, dstr, zeros)
    hs3 = _mid_tc(agg2, dinv, b2.reshape(1, _H), W3)
    agg3 = _agg_kernel(hs3, srcr, dstr, zeros)
    return _final_tc(agg3, dinv, b3.reshape(1, _H), batch_row,
                     Wl, bl.reshape(1, 16))


# speedup vs baseline: 1.2061x; 1.2061x over previous
import functools

import jax
import jax.numpy as jnp
from jax import lax
from jax.experimental import pallas as pl
from jax.experimental.pallas import tpu as pltpu
from jax.experimental.pallas import tpu_sc as plsc

_N = 10000
_NP = 10112
_H = 128
_G = 64
_NC = 2
_NS = 16
_K = 128
_NCHK = 84
_EPW = _K * _NCHK
_EPAD = _EPW * _NC * _NS

_mesh = plsc.VectorSubcoreMesh(core_axis_name="c", subcore_axis_name="s")


def _fill_rows(ref, nrows, width, value):
    @pl.loop(0, nrows)
    def _(r):
        for c in range(0, width, 16):
            ref[r, pl.ds(c, 16)] = jnp.full((16,), value, jnp.float32)


@functools.partial(
    pl.kernel,
    out_type=jax.ShapeDtypeStruct((_NC, _NP, _H), jnp.float32),
    mesh=_mesh,
    scratch_types=[
        pltpu.VMEM((_K,), jnp.int32),
        pltpu.VMEM((_K,), jnp.int32),
        pltpu.VMEM((_K, _H), jnp.float32),
        pltpu.VMEM_SHARED((_NP, _H), jnp.float32),
        pltpu.SemaphoreType.DMA,
        pltpu.SemaphoreType.DMA,
    ],
)
def _deg_kernel(dst_hbm, z_hbm, out_hbm, idx0_v, idx1_v, ones_v, sp,
                semd0, semd1):
    cid = lax.axis_index("c")
    sid = lax.axis_index("s")
    _fill_rows(ones_v, _K, _H, 1.0)

    @pl.when(sid == 0)
    def _():
        pltpu.sync_copy(z_hbm, sp)

    plsc.subcore_barrier()

    pltpu.async_copy(dst_hbm.at[cid, sid, 0], idx0_v, semd0)
    pltpu.async_copy(dst_hbm.at[cid, sid, 1], idx1_v, semd1)

    @pl.loop(0, _NCHK // 2)
    def _(g):
        j0 = 2 * g
        pltpu.make_async_copy(dst_hbm.at[cid, sid, j0], idx0_v,
                              semd0).wait()
        pltpu.sync_copy(ones_v, sp.at[idx0_v], add=True)

        @pl.when(j0 + 2 < _NCHK)
        def _():
            pltpu.async_copy(dst_hbm.at[cid, sid, j0 + 2], idx0_v, semd0)

        pltpu.make_async_copy(dst_hbm.at[cid, sid, j0 + 1], idx1_v,
                              semd1).wait()
        pltpu.sync_copy(ones_v, sp.at[idx1_v], add=True)

        @pl.when(j0 + 3 < _NCHK)
        def _():
            pltpu.async_copy(dst_hbm.at[cid, sid, j0 + 3], idx1_v, semd1)

    plsc.subcore_barrier()

    @pl.when(sid == 1)
    def _():
        pltpu.sync_copy(sp, out_hbm.at[cid])


@functools.partial(
    pl.kernel,
    out_type=jax.ShapeDtypeStruct((_NC, _NP, _H), jnp.float32),
    mesh=_mesh,
    scratch_types=[
        pltpu.VMEM((_K,), jnp.int32),
        pltpu.VMEM((_K,), jnp.int32),
        pltpu.VMEM((_K, _H), jnp.float32),
        pltpu.VMEM_SHARED((_NP, _H), jnp.float32),
        pltpu.SemaphoreType.DMA,
        pltpu.SemaphoreType.DMA,
    ],
)
def _agg_kernel(hs_hbm, src_hbm, dst_hbm, z_hbm, out_hbm,
                idxs_v, idxd_v, rows_v, sp, sem, semi):
    cid = lax.axis_index("c")
    sid = lax.axis_index("s")

    @pl.when(sid == 0)
    def _():
        pltpu.sync_copy(z_hbm, sp)

    plsc.subcore_barrier()

    pltpu.async_copy(src_hbm.at[cid, sid, 0], idxs_v, semi)

    @pl.loop(0, _NCHK)
    def _(j):
        pltpu.make_async_copy(src_hbm.at[cid, sid, j], idxs_v, semi).wait()
        gat = pltpu.async_copy(hs_hbm.at[idxs_v], rows_v, sem)
        pltpu.sync_copy(dst_hbm.at[cid, sid, j], idxd_v)
        gat.wait()
        nxt = jnp.minimum(j + 1, _NCHK - 1)
        pltpu.async_copy(src_hbm.at[cid, sid, nxt], idxs_v, semi)
        pltpu.sync_copy(rows_v, sp.at[idxd_v], add=True)

    pltpu.make_async_copy(src_hbm.at[cid, sid, _NCHK - 1], idxs_v,
                          semi).wait()
    plsc.subcore_barrier()

    @pl.when(sid == 1)
    def _():
        pltpu.sync_copy(sp, out_hbm.at[cid])


def _first_body(x_ref, w_ref, deg_ref, hs_ref, dinv_ref):
    deg = deg_ref[0, :, 0:1] + deg_ref[1, :, 0:1]
    rows = lax.broadcasted_iota(jnp.int32, (_NP, 1), 0)
    dinv = jnp.where(rows < _N, lax.rsqrt(jnp.maximum(deg, 1e-12)), 0.0)
    m = jnp.dot(x_ref[...], w_ref[...],
                preferred_element_type=jnp.float32,
                precision=lax.Precision.HIGHEST)
    hs_ref[...] = dinv * m
    dinv_ref[...] = dinv


_first_tc = pl.pallas_call(
    _first_body,
    out_shape=(jax.ShapeDtypeStruct((_NP, _H), jnp.float32),
               jax.ShapeDtypeStruct((_NP, 1), jnp.float32)),
)


def _mid_body(agg_ref, dinv_ref, b_ref, w_ref, hs_ref):
    agg = agg_ref[0] + agg_ref[1]
    t = jnp.maximum(dinv_ref[...] * agg + b_ref[...], 0.0)
    m = jnp.dot(t, w_ref[...], preferred_element_type=jnp.float32,
                precision=lax.Precision.HIGHEST)
    hs_ref[...] = dinv_ref[...] * m


_mid_tc = pl.pallas_call(
    _mid_body,
    out_shape=jax.ShapeDtypeStruct((_NP, _H), jnp.float32),
)


def _final_body(agg_ref, dinv_ref, b_ref, batch_ref, wl_ref, bl_ref, out_ref):
    h3 = dinv_ref[...] * (agg_ref[0] + agg_ref[1]) + b_ref[...]
    gids = lax.broadcasted_iota(jnp.int32, (_G, _NP), 0)
    oh = (gids == batch_ref[...]).astype(jnp.float32)
    sums = jnp.dot(oh, h3, preferred_element_type=jnp.float32,
                   precision=lax.Precision.HIGHEST)
    cnt = jnp.sum(oh, axis=1, keepdims=True)
    pooled = sums / jnp.maximum(cnt, 1.0)
    out_ref[...] = (jnp.dot(pooled, wl_ref[...],
                            preferred_element_type=jnp.float32,
                            precision=lax.Precision.HIGHEST)
                    + bl_ref[...])


_final_tc = pl.pallas_call(
    _final_body,
    out_shape=jax.ShapeDtypeStruct((_G, 16), jnp.float32),
)


def kernel(x, edge_index, batch, W1, b1, W2, b2, W3, b3, Wl, bl):
    loop = jnp.arange(_N, dtype=jnp.int32)
    src = jnp.concatenate([edge_index[0].astype(jnp.int32), loop])
    dst = jnp.concatenate([edge_index[1].astype(jnp.int32), loop])
    pad = jnp.full((_EPAD - src.shape[0],), _N, dtype=jnp.int32)
    srcr = jnp.concatenate([src, pad]).reshape(_NC, _NS, _NCHK, _K)
    dstr = jnp.concatenate([dst, pad]).reshape(_NC, _NS, _NCHK, _K)
    x_pad = jnp.pad(x, ((0, _NP - _N), (0, 0)))
    batch_row = jnp.pad(batch.astype(jnp.int32), (0, _NP - _N),
                        constant_values=_G).reshape(1, _NP)
    zeros = jnp.zeros((_NP, _H), jnp.float32)

    degp = _deg_kernel(dstr, zeros)
    hs1, dinv = _first_tc(x_pad, W1, degp)
    agg1 = _agg_kernel(hs1, srcr, dstr, zeros)
    hs2 = _mid_tc(agg1, dinv, b1.reshape(1, _H), W2)
    agg2 = _agg_kernel(hs2, srcr, dstr, zeros)
    hs3 = _mid_tc(agg2, dinv, b2.reshape(1, _H), W3)
    agg3 = _agg_kernel(hs3, srcr, dstr, zeros)
    return _final_tc(agg3, dinv, b3.reshape(1, _H), batch_row,
                     Wl, bl.reshape(1, 16))


# speedup vs baseline: 2.7333x; 2.2663x over previous
import functools

import jax
import jax.numpy as jnp
from jax import lax
from jax.experimental import pallas as pl
from jax.experimental.pallas import tpu as pltpu
from jax.experimental.pallas import tpu_sc as plsc

_N = 10000
_NP = 10112
_H = 128
_G = 64
_NC = 2
_NS = 16
_K = 128
_NCHK = 81
_EPW = _K * _NCHK
_EPAD = _EPW * _NC * _NS

_mesh = plsc.VectorSubcoreMesh(core_axis_name="c", subcore_axis_name="s")


def _fill_rows(ref, nrows, width, value):
    @pl.loop(0, nrows)
    def _(r):
        for c in range(0, width, 16):
            ref[r, pl.ds(c, 16)] = jnp.full((16,), value, jnp.float32)


@functools.partial(
    pl.kernel,
    out_type=jax.ShapeDtypeStruct((_NC, _NP, _H), jnp.float32),
    mesh=_mesh,
    scratch_types=[
        pltpu.VMEM((_K,), jnp.int32),
        pltpu.VMEM((_K, _H), jnp.float32),
        pltpu.VMEM_SHARED((_NP, _H), jnp.float32),
    ],
)
def _deg_kernel(dst_hbm, z_hbm, out_hbm, idx_v, ones_v, sp):
    cid = lax.axis_index("c")
    sid = lax.axis_index("s")
    _fill_rows(ones_v, _K, _H, 1.0)

    @pl.when(sid == 0)
    def _():
        pltpu.sync_copy(z_hbm, sp)

    plsc.subcore_barrier()

    @pl.loop(0, _NCHK)
    def _(j):
        pltpu.sync_copy(dst_hbm.at[cid, sid, j], idx_v)
        pltpu.sync_copy(ones_v, sp.at[idx_v], add=True)

    plsc.subcore_barrier()

    @pl.when(sid == 1)
    def _():
        pltpu.sync_copy(sp, out_hbm.at[cid])


@functools.partial(
    pl.kernel,
    out_type=jax.ShapeDtypeStruct((_NC, _NP, _H), jnp.float32),
    mesh=_mesh,
    scratch_types=[
        pltpu.VMEM((_K,), jnp.int32),
        pltpu.VMEM((_K,), jnp.int32),
        pltpu.VMEM((_K, _H), jnp.float32),
        pltpu.VMEM_SHARED((_NP, _H), jnp.float32),
        pltpu.SemaphoreType.DMA,
    ],
)
def _agg_kernel(hs_hbm, src_hbm, dst_hbm, z_hbm, out_hbm,
                idxs_v, idxd_v, rows_v, sp, sem):
    cid = lax.axis_index("c")
    sid = lax.axis_index("s")

    @pl.when(sid == 0)
    def _():
        pltpu.sync_copy(z_hbm, sp)

    plsc.subcore_barrier()

    @pl.loop(0, _NCHK)
    def _(j):
        pltpu.sync_copy(src_hbm.at[cid, sid, j], idxs_v)
        pltpu.sync_copy(dst_hbm.at[cid, sid, j], idxd_v)
        pltpu.async_copy(hs_hbm.at[idxs_v], rows_v, sem).wait()
        pltpu.sync_copy(rows_v, sp.at[idxd_v], add=True)

    plsc.subcore_barrier()

    @pl.when(sid == 1)
    def _():
        pltpu.sync_copy(sp, out_hbm.at[cid])


def _first_body(x_ref, w_ref, deg_ref, hs_ref, dinv_ref):
    deg = deg_ref[0, :, 0:1] + deg_ref[1, :, 0:1]
    rows = lax.broadcasted_iota(jnp.int32, (_NP, 1), 0)
    dinv = jnp.where(rows < _N, lax.rsqrt(jnp.maximum(deg, 1e-12)), 0.0)
    m = jnp.dot(x_ref[...], w_ref[...],
                preferred_element_type=jnp.float32,
                precision=lax.Precision.HIGHEST)
    hs_ref[...] = dinv * m
    dinv_ref[...] = dinv


_first_tc = pl.pallas_call(
    _first_body,
    out_shape=(jax.ShapeDtypeStruct((_NP, _H), jnp.float32),
               jax.ShapeDtypeStruct((_NP, 1), jnp.float32)),
)


def _mid_body(agg_ref, dinv_ref, b_ref, w_ref, hs_ref):
    agg = agg_ref[0] + agg_ref[1]
    t = jnp.maximum(dinv_ref[...] * agg + b_ref[...], 0.0)
    m = jnp.dot(t, w_ref[...], preferred_element_type=jnp.float32,
                precision=lax.Precision.HIGHEST)
    hs_ref[...] = dinv_ref[...] * m


_mid_tc = pl.pallas_call(
    _mid_body,
    out_shape=jax.ShapeDtypeStruct((_NP, _H), jnp.float32),
)


def _final_body(agg_ref, dinv_ref, b_ref, batch_ref, wl_ref, bl_ref, out_ref):
    h3 = dinv_ref[...] * (agg_ref[0] + agg_ref[1]) + b_ref[...]
    gids = lax.broadcasted_iota(jnp.int32, (_G, _NP), 0)
    oh = (gids == batch_ref[...]).astype(jnp.float32)
    sums = jnp.dot(oh, h3, preferred_element_type=jnp.float32,
                   precision=lax.Precision.HIGHEST)
    cnt = jnp.sum(oh, axis=1, keepdims=True)
    pooled = sums / jnp.maximum(cnt, 1.0)
    out_ref[...] = (jnp.dot(pooled, wl_ref[...],
                            preferred_element_type=jnp.float32,
                            precision=lax.Precision.HIGHEST)
                    + bl_ref[...])


_final_tc = pl.pallas_call(
    _final_body,
    out_shape=jax.ShapeDtypeStruct((_G, 16), jnp.float32),
)


def kernel(x, edge_index, batch, W1, b1, W2, b2, W3, b3, Wl, bl):
    loop = jnp.arange(_N, dtype=jnp.int32)
    src = jnp.concatenate([edge_index[0].astype(jnp.int32), loop])
    dst = jnp.concatenate([edge_index[1].astype(jnp.int32), loop])
    pad = jnp.full((_EPAD - src.shape[0],), _N, dtype=jnp.int32)
    srcr = jnp.concatenate([src, pad]).reshape(_NC, _NS, _NCHK, _K)
    dstr = jnp.concatenate([dst, pad]).reshape(_NC, _NS, _NCHK, _K)
    x_pad = jnp.pad(x, ((0, _NP - _N), (0, 0)))
    batch_row = jnp.pad(batch.astype(jnp.int32), (0, _NP - _N),
                        constant_values=_G).reshape(1, _NP)
    zeros = jnp.zeros((_NP, _H), jnp.float32)

    degp = _deg_kernel(dstr, zeros)
    hs1, dinv = _first_tc(x_pad, W1, degp)
    agg1 = _agg_kernel(hs1, srcr, dstr, zeros)
    hs2 = _mid_tc(agg1, dinv, b1.reshape(1, _H), W2)
    agg2 = _agg_kernel(hs2, srcr, dstr, zeros)
    hs3 = _mid_tc(agg2, dinv, b2.reshape(1, _H), W3)
    agg3 = _agg_kernel(hs3, srcr, dstr, zeros)
    return _final_tc(agg3, dinv, b3.reshape(1, _H), batch_row,
                     Wl, bl.reshape(1, 16))


# speedup vs baseline: 3.0028x; 1.0986x over previous
import functools

import jax
import jax.numpy as jnp
from jax import lax
from jax.experimental import pallas as pl
from jax.experimental.pallas import tpu as pltpu
from jax.experimental.pallas import tpu_sc as plsc

_N = 10000
_NP = 10112
_H = 128
_G = 64
_NC = 2
_NS = 16
_K = 128
_NCHK = 81
_EPW = _K * _NCHK
_EPAD = _EPW * _NC * _NS

_mesh = plsc.VectorSubcoreMesh(core_axis_name="c", subcore_axis_name="s")


def _fill_rows(ref, nrows, width, value):
    @pl.loop(0, nrows)
    def _(r):
        for c in range(0, width, 16):
            ref[r, pl.ds(c, 16)] = jnp.full((16,), value, jnp.float32)


@functools.partial(
    pl.kernel,
    out_type=jax.ShapeDtypeStruct((_NC, _NP, _H), jnp.float32),
    mesh=_mesh,
    scratch_types=[
        pltpu.VMEM((_K,), jnp.int32),
        pltpu.VMEM((_K, _H), jnp.float32),
        pltpu.VMEM_SHARED((_NP, _H), jnp.float32),
    ],
)
def _deg_kernel(dst_hbm, z_hbm, out_hbm, idx_v, ones_v, sp):
    cid = lax.axis_index("c")
    sid = lax.axis_index("s")
    _fill_rows(ones_v, _K, _H, 1.0)

    @pl.when(sid == 0)
    def _():
        pltpu.sync_copy(z_hbm, sp)

    plsc.subcore_barrier()

    @pl.loop(0, _NCHK)
    def _(j):
        pltpu.sync_copy(dst_hbm.at[cid, sid, j], idx_v)
        pltpu.sync_copy(ones_v, sp.at[idx_v], add=True)

    plsc.subcore_barrier()

    @pl.when(sid == 1)
    def _():
        pltpu.sync_copy(sp, out_hbm.at[cid])


@functools.partial(
    pl.kernel,
    out_type=jax.ShapeDtypeStruct((_NC, _NP, _H), jnp.float32),
    mesh=_mesh,
    scratch_types=[
        pltpu.VMEM((_K,), jnp.int32),
        pltpu.VMEM((_K,), jnp.int32),
        pltpu.VMEM((_K, _H), jnp.float32),
        pltpu.VMEM_SHARED((_NP, _H), jnp.float32),
        pltpu.SemaphoreType.DMA,
    ],
)
def _agg_kernel(hs_hbm, src_hbm, dst_hbm, z_hbm, out_hbm,
                idxs_v, idxd_v, rows_v, sp, sem):
    cid = lax.axis_index("c")
    sid = lax.axis_index("s")

    @pl.when(sid == 0)
    def _():
        pltpu.sync_copy(z_hbm, sp)

    plsc.subcore_barrier()

    @pl.loop(0, _NCHK)
    def _(j):
        pltpu.sync_copy(src_hbm.at[cid, sid, j], idxs_v)
        gat = pltpu.async_copy(hs_hbm.at[idxs_v], rows_v, sem)
        pltpu.sync_copy(dst_hbm.at[cid, sid, j], idxd_v)
        gat.wait()
        pltpu.sync_copy(rows_v, sp.at[idxd_v], add=True)

    plsc.subcore_barrier()

    @pl.when(sid == 1)
    def _():
        pltpu.sync_copy(sp, out_hbm.at[cid])


def _first_body(x_ref, w_ref, deg_ref, hs_ref, dinv_ref):
    deg = deg_ref[0, :, 0:1] + deg_ref[1, :, 0:1]
    rows = lax.broadcasted_iota(jnp.int32, (_NP, 1), 0)
    dinv = jnp.where(rows < _N, lax.rsqrt(jnp.maximum(deg, 1e-12)), 0.0)
    m = jnp.dot(x_ref[...], w_ref[...],
                preferred_element_type=jnp.float32,
                precision=lax.Precision.HIGHEST)
    hs_ref[...] = dinv * m
    dinv_ref[...] = dinv


_first_tc = pl.pallas_call(
    _first_body,
    out_shape=(jax.ShapeDtypeStruct((_NP, _H), jnp.float32),
               jax.ShapeDtypeStruct((_NP, 1), jnp.float32)),
)


def _mid_body(agg_ref, dinv_ref, b_ref, w_ref, hs_ref):
    agg = agg_ref[0] + agg_ref[1]
    t = jnp.maximum(dinv_ref[...] * agg + b_ref[...], 0.0)
    m = jnp.dot(t, w_ref[...], preferred_element_type=jnp.float32,
                precision=lax.Precision.HIGHEST)
    hs_ref[...] = dinv_ref[...] * m


_mid_tc = pl.pallas_call(
    _mid_body,
    out_shape=jax.ShapeDtypeStruct((_NP, _H), jnp.float32),
)


def _final_body(agg_ref, dinv_ref, b_ref, batch_ref, wl_ref, bl_ref, out_ref):
    h3 = dinv_ref[...] * (agg_ref[0] + agg_ref[1]) + b_ref[...]
    gids = lax.broadcasted_iota(jnp.int32, (_G, _NP), 0)
    oh = (gids == batch_ref[...]).astype(jnp.float32)
    sums = jnp.dot(oh, h3, preferred_element_type=jnp.float32,
                   precision=lax.Precision.HIGHEST)
    cnt = jnp.sum(oh, axis=1, keepdims=True)
    pooled = sums / jnp.maximum(cnt, 1.0)
    out_ref[...] = (jnp.dot(pooled, wl_ref[...],
                            preferred_element_type=jnp.float32,
                            precision=lax.Precision.HIGHEST)
                    + bl_ref[...])


_final_tc = pl.pallas_call(
    _final_body,
    out_shape=jax.ShapeDtypeStruct((_G, 16), jnp.float32),
)


def kernel(x, edge_index, batch, W1, b1, W2, b2, W3, b3, Wl, bl):
    loop = jnp.arange(_N, dtype=jnp.int32)
    src = jnp.concatenate([edge_index[0].astype(jnp.int32), loop])
    dst = jnp.concatenate([edge_index[1].astype(jnp.int32), loop])
    pad = jnp.full((_EPAD - src.shape[0],), _N, dtype=jnp.int32)
    srcr = jnp.concatenate([src, pad]).reshape(_NC, _NS, _NCHK, _K)
    dstr = jnp.concatenate([dst, pad]).reshape(_NC, _NS, _NCHK, _K)
    x_pad = jnp.pad(x, ((0, _NP - _N), (0, 0)))
    batch_row = jnp.pad(batch.astype(jnp.int32), (0, _NP - _N),
                        constant_values=_G).reshape(1, _NP)
    zeros = jnp.zeros((_NP, _H), jnp.float32)

    degp = _deg_kernel(dstr, zeros)
    hs1, dinv = _first_tc(x_pad, W1, degp)
    agg1 = _agg_kernel(hs1, srcr, dstr, zeros)
    hs2 = _mid_tc(agg1, dinv, b1.reshape(1, _H), W2)
    agg2 = _agg_kernel(hs2, srcr, dstr, zeros)
    hs3 = _mid_tc(agg2, dinv, b2.reshape(1, _H), W3)
    agg3 = _agg_kernel(hs3, srcr, dstr, zeros)
    return _final_tc(agg3, dinv, b3.reshape(1, _H), batch_row,
                     Wl, bl.reshape(1, 16))


# speedup vs baseline: 3.0977x; 1.0316x over previous
import functools

import jax
import jax.numpy as jnp
from jax import lax
from jax.experimental import pallas as pl
from jax.experimental.pallas import tpu as pltpu
from jax.experimental.pallas import tpu_sc as plsc

_N = 10000
_NP = 10112
_H = 128
_G = 64
_NC = 2
_NS = 16
_K = 128
_NCHK = 81
_EPW = _K * _NCHK
_EPAD = _EPW * _NC * _NS

_mesh = plsc.VectorSubcoreMesh(core_axis_name="c", subcore_axis_name="s")


def _fill_rows(ref, nrows, width, value):
    @pl.loop(0, nrows)
    def _(r):
        for c in range(0, width, 16):
            ref[r, pl.ds(c, 16)] = jnp.full((16,), value, jnp.float32)


@functools.partial(
    pl.kernel,
    out_type=jax.ShapeDtypeStruct((_NC, _NP, _H), jnp.float32),
    mesh=_mesh,
    scratch_types=[
        pltpu.VMEM((_K,), jnp.int32),
        pltpu.VMEM((_K,), jnp.int32),
        pltpu.VMEM((_K, _H), jnp.float32),
        pltpu.VMEM_SHARED((_NP, _H), jnp.float32),
        pltpu.SemaphoreType.DMA,
        pltpu.SemaphoreType.DMA,
    ],
)
def _deg_kernel(dst_hbm, z_hbm, out_hbm, idx0_v, idx1_v, ones_v, sp,
                semd0, semd1):
    cid = lax.axis_index("c")
    sid = lax.axis_index("s")
    _fill_rows(ones_v, _K, _H, 1.0)

    @pl.when(sid == 0)
    def _():
        pltpu.sync_copy(z_hbm, sp)

    plsc.subcore_barrier()

    pltpu.async_copy(dst_hbm.at[cid, sid, 0], idx0_v, semd0)
    pltpu.async_copy(dst_hbm.at[cid, sid, 1], idx1_v, semd1)

    @pl.loop(0, (_NCHK + 1) // 2)
    def _(g):
        j0 = 2 * g
        pltpu.make_async_copy(dst_hbm.at[cid, sid, j0], idx0_v,
                              semd0).wait()
        pltpu.sync_copy(ones_v, sp.at[idx0_v], add=True)

        @pl.when(j0 + 2 < _NCHK)
        def _():
            pltpu.async_copy(dst_hbm.at[cid, sid, j0 + 2], idx0_v, semd0)

        @pl.when(j0 + 1 < _NCHK)
        def _():
            pltpu.make_async_copy(dst_hbm.at[cid, sid, j0 + 1], idx1_v,
                                  semd1).wait()
            pltpu.sync_copy(ones_v, sp.at[idx1_v], add=True)

        @pl.when(j0 + 3 < _NCHK)
        def _():
            pltpu.async_copy(dst_hbm.at[cid, sid, j0 + 3], idx1_v, semd1)

    plsc.subcore_barrier()

    @pl.when(sid == 1)
    def _():
        pltpu.sync_copy(sp, out_hbm.at[cid])


@functools.partial(
    pl.kernel,
    out_type=jax.ShapeDtypeStruct((_NC, _NP, _H), jnp.float32),
    mesh=_mesh,
    scratch_types=[
        pltpu.VMEM((_K,), jnp.int32),
        pltpu.VMEM((_K,), jnp.int32),
        pltpu.VMEM((_K, _H), jnp.float32),
        pltpu.VMEM_SHARED((_NP, _H), jnp.float32),
        pltpu.SemaphoreType.DMA,
    ],
)
def _agg_kernel(hs_hbm, src_hbm, dst_hbm, z_hbm, out_hbm,
                idxs_v, idxd_v, rows_v, sp, sem):
    cid = lax.axis_index("c")
    sid = lax.axis_index("s")

    @pl.when(sid == 0)
    def _():
        pltpu.sync_copy(z_hbm, sp)

    plsc.subcore_barrier()

    @pl.loop(0, _NCHK)
    def _(j):
        pltpu.sync_copy(src_hbm.at[cid, sid, j], idxs_v)
        gat = pltpu.async_copy(hs_hbm.at[idxs_v], rows_v, sem)
        pltpu.sync_copy(dst_hbm.at[cid, sid, j], idxd_v)
        gat.wait()
        pltpu.sync_copy(rows_v, sp.at[idxd_v], add=True)

    plsc.subcore_barrier()

    @pl.when(sid == 1)
    def _():
        pltpu.sync_copy(sp, out_hbm.at[cid])


def _first_body(x_ref, w_ref, deg_ref, hs_ref, dinv_ref):
    deg = deg_ref[0, :, 0:1] + deg_ref[1, :, 0:1]
    rows = lax.broadcasted_iota(jnp.int32, (_NP, 1), 0)
    dinv = jnp.where(rows < _N, lax.rsqrt(jnp.maximum(deg, 1e-12)), 0.0)
    m = jnp.dot(x_ref[...], w_ref[...],
                preferred_element_type=jnp.float32,
                precision=lax.Precision.HIGHEST)
    hs_ref[...] = dinv * m
    dinv_ref[...] = dinv


_first_tc = pl.pallas_call(
    _first_body,
    out_shape=(jax.ShapeDtypeStruct((_NP, _H), jnp.float32),
               jax.ShapeDtypeStruct((_NP, 1), jnp.float32)),
)


def _mid_body(agg_ref, dinv_ref, b_ref, w_ref, hs_ref):
    agg = agg_ref[0] + agg_ref[1]
    t = jnp.maximum(dinv_ref[...] * agg + b_ref[...], 0.0)
    m = jnp.dot(t, w_ref[...], preferred_element_type=jnp.float32,
                precision=lax.Precision.HIGHEST)
    hs_ref[...] = dinv_ref[...] * m


_mid_tc = pl.pallas_call(
    _mid_body,
    out_shape=jax.ShapeDtypeStruct((_NP, _H), jnp.float32),
)


def _final_body(agg_ref, dinv_ref, b_ref, batch_ref, wl_ref, bl_ref, out_ref):
    h3 = dinv_ref[...] * (agg_ref[0] + agg_ref[1]) + b_ref[...]
    gids = lax.broadcasted_iota(jnp.int32, (_G, _NP), 0)
    oh = (gids == batch_ref[...]).astype(jnp.float32)
    sums = jnp.dot(oh, h3, preferred_element_type=jnp.float32,
                   precision=lax.Precision.HIGHEST)
    cnt = jnp.sum(oh, axis=1, keepdims=True)
    pooled = sums / jnp.maximum(cnt, 1.0)
    out_ref[...] = (jnp.dot(pooled, wl_ref[...],
                            preferred_element_type=jnp.float32,
                            precision=lax.Precision.HIGHEST)
                    + bl_ref[...])


_final_tc = pl.pallas_call(
    _final_body,
    out_shape=jax.ShapeDtypeStruct((_G, 16), jnp.float32),
)


def kernel(x, edge_index, batch, W1, b1, W2, b2, W3, b3, Wl, bl):
    loop = jnp.arange(_N, dtype=jnp.int32)
    src = jnp.concatenate([edge_index[0].astype(jnp.int32), loop])
    dst = jnp.concatenate([edge_index[1].astype(jnp.int32), loop])
    pad = jnp.full((_EPAD - src.shape[0],), _N, dtype=jnp.int32)
    srcr = jnp.concatenate([src, pad]).reshape(_NC, _NS, _NCHK, _K)
    dstr = jnp.concatenate([dst, pad]).reshape(_NC, _NS, _NCHK, _K)
    x_pad = jnp.pad(x, ((0, _NP - _N), (0, 0)))
    batch_row = jnp.pad(batch.astype(jnp.int32), (0, _NP - _N),
                        constant_values=_G).reshape(1, _NP)
    zeros = jnp.zeros((_NP, _H), jnp.float32)

    degp = _deg_kernel(dstr, zeros)
    hs1, dinv = _first_tc(x_pad, W1, degp)
    agg1 = _agg_kernel(hs1, srcr, dstr, zeros)
    hs2 = _mid_tc(agg1, dinv, b1.reshape(1, _H), W2)
    agg2 = _agg_kernel(hs2, srcr, dstr, zeros)
    hs3 = _mid_tc(agg2, dinv, b2.reshape(1, _H), W3)
    agg3 = _agg_kernel(hs3, srcr, dstr, zeros)
    return _final_tc(agg3, dinv, b3.reshape(1, _H), batch_row,
                     Wl, bl.reshape(1, 16))
